# Initial kernel scaffold; baseline (speedup 1.0000x reference)
import jax
import jax.numpy as jnp
from jax.experimental import pallas as pl


def kernel(x, edge_index, edge_attr, smiles, batch, is_supervised, W_in, b_in, W_out, b_out, W_p1, b_p1, W_p2, b_p2, W_h1, b_h1, W_h2, b_h2):
    raise NotImplementedError("write your pallas kernel here")


# speedup vs baseline: 1.1227x; 1.1227x over previous
import functools

import jax
import jax.numpy as jnp
from jax import lax
from jax.experimental import pallas as pl
from jax.experimental.pallas import tpu as pltpu
from jax.experimental.pallas import tpu_sc as plsc

N = 10000
E = 160000
D = 256
DE = 16
G = 512
HALF = 128
NP = 10240
STRIPE = NP // 16
DUMMY = N
EP = 163840
CHUNKS = 80
EA_CHUNKS = 40
BLK = 2048
NBLK = NP // BLK

_f32 = jnp.float32


def _make_sc_spmv(do_ea: bool):
    mesh = plsc.VectorSubcoreMesh(core_axis_name="c", subcore_axis_name="s",
                                  num_cores=2, num_subcores=16)
    out_type = [jax.ShapeDtypeStruct((2, NP, HALF), _f32)]
    scratch = [
        pltpu.VMEM_SHARED((NP, HALF), _f32),
        pltpu.VMEM((CHUNKS // 5, 128), jnp.int32),
        pltpu.VMEM((CHUNKS // 5, 128), jnp.int32),
        pltpu.VMEM((128, HALF), _f32),
    ]
    if do_ea:
        out_type.append(jax.ShapeDtypeStruct((2, NP, DE), _f32))
        scratch += [
            pltpu.VMEM_SHARED((NP, DE), _f32),
            pltpu.VMEM((EA_CHUNKS // 5, 128), jnp.int32),
            pltpu.VMEM((128, DE), _f32),
        ]

    def body(*refs):
        if do_ea:
            (table, srcs, dsts, ea_dst, ea_val, zeros_ea,
             out, ea_out, acc, src_v, dst_v, gbuf,
             ea_acc, ea_dst_v, ea_buf) = refs
            init = table
        else:
            (table, init, srcs, dsts,
             out, acc, src_v, dst_v, gbuf) = refs
        c = lax.axis_index("c")
        t = lax.axis_index("s")
        r0 = t * STRIPE
        def init_chunk(h, carry):
            rh = r0 + h * 128
            pltpu.sync_copy(init.at[c, pl.ds(rh, 128)], gbuf)
            pltpu.sync_copy(gbuf, acc.at[pl.ds(rh, 128)])
            if do_ea:
                pltpu.sync_copy(zeros_ea.at[pl.ds(rh, 128)], ea_buf)
                pltpu.sync_copy(ea_buf, ea_acc.at[pl.ds(rh, 128)])
            return carry

        lax.fori_loop(0, STRIPE // 128, init_chunk, 0, unroll=False)
        plsc.subcore_barrier()

        tbl = table.at[c]

        QC = CHUNKS // 5

        def stage_and_scan(q, carry):
            pltpu.sync_copy(srcs.at[t, pl.ds(q * QC, QC)], src_v)
            pltpu.sync_copy(dsts.at[t, pl.ds(q * QC, QC)], dst_v)

            def spmv_chunk(j, carry2):
                pltpu.sync_copy(tbl.at[src_v.at[j]], gbuf)
                pltpu.sync_copy(gbuf, acc.at[dst_v.at[j]], add=True)
                return carry2

            return lax.fori_loop(0, QC, spmv_chunk, carry)

        lax.fori_loop(0, 5, stage_and_scan, 0, unroll=False)

        if do_ea:
            EQC = EA_CHUNKS // 5

            def ea_stage_and_scan(q, carry):
                pltpu.sync_copy(ea_dst.at[c, t, pl.ds(q * EQC, EQC)], ea_dst_v)

                def ea_chunk(j, carry2):
                    pltpu.sync_copy(
                        ea_val.at[c, t, pl.ds((q * EQC + j) * 128, 128)], ea_buf)
                    pltpu.sync_copy(ea_buf, ea_acc.at[ea_dst_v.at[j]], add=True)
                    return carry2

                return lax.fori_loop(0, EQC, ea_chunk, carry)

            lax.fori_loop(0, 5, ea_stage_and_scan, 0, unroll=False)

        plsc.subcore_barrier()

        def out_chunk(h, carry):
            rh = r0 + h * 128
            pltpu.sync_copy(acc.at[pl.ds(rh, 128)], gbuf)
            pltpu.sync_copy(gbuf, out.at[c, pl.ds(rh, 128)])
            if do_ea:
                pltpu.sync_copy(ea_acc.at[pl.ds(rh, 128)], ea_buf)
                pltpu.sync_copy(ea_buf, ea_out.at[c, pl.ds(rh, 128)])
            return carry

        lax.fori_loop(0, STRIPE // 128, out_chunk, 0, unroll=False)

    return pl.kernel(body, out_type=tuple(out_type), mesh=mesh,
                     scratch_types=scratch)


@functools.lru_cache(maxsize=None)
def _sc_spmv(do_ea: bool):
    return _make_sc_spmv(do_ea)


def _tc_conv_body(s_ref, ea_ref, w1t_ref, w2t_ref, b_ref, y_ref, b1_ref):
    s = jnp.concatenate([s_ref[0], s_ref[1]], axis=-1)
    ea = ea_ref[0] + ea_ref[1]
    cterm = jnp.dot(ea, w2t_ref[...], preferred_element_type=_f32, precision=lax.Precision.HIGHEST) + b_ref[...]
    x1 = jnp.dot(s, w1t_ref[...], preferred_element_type=_f32, precision=lax.Precision.HIGHEST) + cterm
    y1 = jnp.dot(x1, w1t_ref[...], preferred_element_type=_f32, precision=lax.Precision.HIGHEST)
    b1 = y1 + cterm
    y_ref[0] = y1[:, :HALF]
    y_ref[1] = y1[:, HALF:]
    b1_ref[0] = b1[:, :HALF]
    b1_ref[1] = b1[:, HALF:]


_tc_conv = pl.pallas_call(
    _tc_conv_body,
    grid=(NBLK,),
    in_specs=[
        pl.BlockSpec((2, BLK, HALF), lambda i: (0, i, 0)),
        pl.BlockSpec((2, BLK, DE), lambda i: (0, i, 0)),
        pl.BlockSpec((D, D), lambda i: (0, 0)),
        pl.BlockSpec((DE, D), lambda i: (0, 0)),
        pl.BlockSpec((1, D), lambda i: (0, 0)),
    ],
    out_specs=[
        pl.BlockSpec((2, BLK, HALF), lambda i: (0, i, 0)),
        pl.BlockSpec((2, BLK, HALF), lambda i: (0, i, 0)),
    ],
    out_shape=[
        jax.ShapeDtypeStruct((2, NP, HALF), _f32),
        jax.ShapeDtypeStruct((2, NP, HALF), _f32),
    ],
)


def _tc_pool_body(x_ref, batch_ref, wot_ref, b_ref, mf_ref):
    x2 = jnp.concatenate([x_ref[0], x_ref[1]], axis=-1)
    l = jnp.dot(x2, wot_ref[...], preferred_element_type=_f32, precision=lax.Precision.HIGHEST) + b_ref[...]
    m = jnp.max(l, axis=1, keepdims=True)
    p = jnp.exp(l - m)
    a = p / jnp.sum(p, axis=1, keepdims=True)
    bt = batch_ref[...]
    gid = lax.broadcasted_iota(jnp.int32, (1, G), 1)
    oh = (bt == gid).astype(_f32)
    part = lax.dot_general(oh, a, (((0,), (0,)), ((), ())),
                           preferred_element_type=_f32, precision=lax.Precision.HIGHEST)
    i = pl.program_id(0)

    @pl.when(i == 0)
    def _():
        mf_ref[...] = part

    @pl.when(i > 0)
    def _():
        mf_ref[...] += part


_tc_pool = pl.pallas_call(
    _tc_pool_body,
    grid=(NBLK,),
    in_specs=[
        pl.BlockSpec((2, BLK, HALF), lambda i: (0, i, 0)),
        pl.BlockSpec((BLK, 1), lambda i: (i, 0)),
        pl.BlockSpec((D, 512), lambda i: (0, 0)),
        pl.BlockSpec((1, 512), lambda i: (0, 0)),
    ],
    out_specs=pl.BlockSpec((G, 512), lambda i: (0, 0)),
    out_shape=jax.ShapeDtypeStruct((G, 512), _f32),
)


def _tc_readout_body(mf_ref, wh1_ref, bh1_ref, wh2_ref, bh2_ref,
                     wp1_ref, bp1_ref, wp2_ref, bp2_ref, o_ref, z_ref):
    mf = mf_ref[...]
    h1 = jnp.maximum(
        jnp.dot(mf, wh1_ref[...], preferred_element_type=_f32, precision=lax.Precision.HIGHEST) + bh1_ref[...], 0.0)
    z_ref[...] = jnp.dot(h1, wh2_ref[...], preferred_element_type=_f32, precision=lax.Precision.HIGHEST) + bh2_ref[...]
    hid = jnp.dot(mf, wp1_ref[...], preferred_element_type=_f32, precision=lax.Precision.HIGHEST) + bp1_ref[...]
    o = jnp.dot(hid, wp2_ref[...], preferred_element_type=_f32, precision=lax.Precision.HIGHEST) + bp2_ref[...]
    o_ref[...] = jax.nn.sigmoid(o)


_tc_readout = pl.pallas_call(
    _tc_readout_body,
    out_shape=[
        jax.ShapeDtypeStruct((G, 1), _f32),
        jax.ShapeDtypeStruct((G, 1), _f32),
    ],
)


def kernel(x, edge_index, edge_attr, smiles, batch, is_supervised,
           W_in, b_in, W_out, b_out, W_p1, b_p1, W_p2, b_p2,
           W_h1, b_h1, W_h2, b_h2):
    del smiles, is_supervised
    xp = jnp.zeros((NP, D), _f32).at[:N].set(x)
    xh = xp.reshape(NP, 2, HALF).transpose(1, 0, 2)
    pad = EP - E
    srcp = jnp.concatenate(
        [edge_index[0], jnp.zeros((pad,), jnp.int32)]).reshape(16, CHUNKS, 128)
    dstp = jnp.concatenate(
        [edge_index[1], jnp.full((pad,), DUMMY, jnp.int32)]
    ).reshape(16, CHUNKS, 128)
    ea_dst = jnp.concatenate(
        [edge_index[1], jnp.zeros((pad,), jnp.int32)]
    ).reshape(2, 16, EA_CHUNKS, 128)
    ea_val = jnp.zeros((EP, DE), _f32).at[:E].set(edge_attr)
    ea_val = ea_val.reshape(2, 16, EP // 32, DE)
    zeros_ea = jnp.zeros((NP, DE), _f32)
    batchp = jnp.concatenate(
        [batch, jnp.full((NP - N,), G, jnp.int32)]).reshape(NP, 1)
    w1t = W_in[:, :D].T
    w2t = W_in[:, D:].T

    def _emu(table, init, sp, dp):
        outs = []
        for cc in range(2):
            accv = init[cc].at[dp.reshape(-1)].add(table[cc][sp.reshape(-1)])
            outs.append(accv)
        return jnp.stack(outs)
    def _emu_ea(ed, ev):
        outs = []
        for cc in range(2):
            accv = jnp.zeros((NP, DE), _f32).at[ed[cc].reshape(-1)].add(ev[cc].reshape(-1, DE))
            outs.append(accv)
        return jnp.stack(outs)
    s0 = _emu(xh, xh, srcp, dstp)
    eap = _emu_ea(ea_dst, ea_val)
    y1, b1 = _tc_conv(s0, eap, w1t, w2t, b_in.reshape(1, D))
    x2 = _emu(y1, b1, srcp, dstp)
    mf = _tc_pool(x2, batchp, W_out.T, b_out.reshape(1, 512))
    o, z = _tc_readout(mf, W_h1.T, b_h1.reshape(1, -1), W_h2.T,
                       b_h2.reshape(1, -1), W_p1.T, b_p1.reshape(1, -1),
                       W_p2.T, b_p2.reshape(1, -1))
    return (o, z)


# speedup vs baseline: 2.9610x; 2.6374x over previous
import functools

import jax
import jax.numpy as jnp
from jax import lax
from jax.experimental import pallas as pl
from jax.experimental.pallas import tpu as pltpu
from jax.experimental.pallas import tpu_sc as plsc

N = 10000
E = 160000
D = 256
DE = 16
G = 512
HALF = 128
NP = 10240
STRIPE = NP // 16
DUMMY = N
EP = 163840
CHUNKS = 80
EA_CHUNKS = 40
BLK = 2048
NBLK = NP // BLK

_f32 = jnp.float32


def _make_sc_spmv(do_ea: bool):
    mesh = plsc.VectorSubcoreMesh(core_axis_name="c", subcore_axis_name="s",
                                  num_cores=2, num_subcores=16)
    out_type = [jax.ShapeDtypeStruct((2, NP, HALF), _f32)]
    scratch = [
        pltpu.VMEM_SHARED((NP, HALF), _f32),
        pltpu.VMEM((CHUNKS // 5, 128), jnp.int32),
        pltpu.VMEM((CHUNKS // 5, 128), jnp.int32),
        pltpu.VMEM((128, HALF), _f32),
    ]
    if do_ea:
        out_type.append(jax.ShapeDtypeStruct((2, NP, DE), _f32))
        scratch += [
            pltpu.VMEM_SHARED((NP, DE), _f32),
            pltpu.VMEM((EA_CHUNKS // 5, 128), jnp.int32),
            pltpu.VMEM((128, DE), _f32),
        ]

    def body(*refs):
        if do_ea:
            (table, srcs, dsts, ea_dst, ea_val, zeros_ea,
             out, ea_out, acc, src_v, dst_v, gbuf,
             ea_acc, ea_dst_v, ea_buf) = refs
            init = table
        else:
            (table, init, srcs, dsts,
             out, acc, src_v, dst_v, gbuf) = refs
        c = lax.axis_index("c")
        t = lax.axis_index("s")
        r0 = t * STRIPE
        def init_chunk(h, carry):
            rh = r0 + h * 128
            pltpu.sync_copy(init.at[c, pl.ds(rh, 128)], gbuf)
            pltpu.sync_copy(gbuf, acc.at[pl.ds(rh, 128)])
            if do_ea:
                pltpu.sync_copy(zeros_ea.at[pl.ds(rh, 128)], ea_buf)
                pltpu.sync_copy(ea_buf, ea_acc.at[pl.ds(rh, 128)])
            return carry

        lax.fori_loop(0, STRIPE // 128, init_chunk, 0, unroll=False)
        plsc.subcore_barrier()

        tbl = table.at[c]

        QC = CHUNKS // 5

        def stage_and_scan(q, carry):
            pltpu.sync_copy(srcs.at[t, pl.ds(q * QC, QC)], src_v)
            pltpu.sync_copy(dsts.at[t, pl.ds(q * QC, QC)], dst_v)

            def spmv_chunk(j, carry2):
                pltpu.sync_copy(tbl.at[src_v.at[j]], gbuf)
                pltpu.sync_copy(gbuf, acc.at[dst_v.at[j]], add=True)
                return carry2

            return lax.fori_loop(0, QC, spmv_chunk, carry)

        lax.fori_loop(0, 5, stage_and_scan, 0, unroll=False)

        if do_ea:
            EQC = EA_CHUNKS // 5

            def ea_stage_and_scan(q, carry):
                pltpu.sync_copy(ea_dst.at[c, t, pl.ds(q * EQC, EQC)], ea_dst_v)

                def ea_chunk(j, carry2):
                    pltpu.sync_copy(
                        ea_val.at[c, t, pl.ds((q * EQC + j) * 128, 128)], ea_buf)
                    pltpu.sync_copy(ea_buf, ea_acc.at[ea_dst_v.at[j]], add=True)
                    return carry2

                return lax.fori_loop(0, EQC, ea_chunk, carry)

            lax.fori_loop(0, 5, ea_stage_and_scan, 0, unroll=False)

        plsc.subcore_barrier()

        def out_chunk(h, carry):
            rh = r0 + h * 128
            pltpu.sync_copy(acc.at[pl.ds(rh, 128)], gbuf)
            pltpu.sync_copy(gbuf, out.at[c, pl.ds(rh, 128)])
            if do_ea:
                pltpu.sync_copy(ea_acc.at[pl.ds(rh, 128)], ea_buf)
                pltpu.sync_copy(ea_buf, ea_out.at[c, pl.ds(rh, 128)])
            return carry

        lax.fori_loop(0, STRIPE // 128, out_chunk, 0, unroll=False)

    return pl.kernel(body, out_type=tuple(out_type), mesh=mesh,
                     scratch_types=scratch)


@functools.lru_cache(maxsize=None)
def _sc_spmv(do_ea: bool):
    return _make_sc_spmv(do_ea)


def _tc_conv_body(s_ref, ea_ref, w1t_ref, w2t_ref, b_ref, y_ref, b1_ref):
    s = jnp.concatenate([s_ref[0], s_ref[1]], axis=-1)
    ea = ea_ref[0] + ea_ref[1]
    cterm = jnp.dot(ea, w2t_ref[...], preferred_element_type=_f32, precision=lax.Precision.HIGHEST) + b_ref[...]
    x1 = jnp.dot(s, w1t_ref[...], preferred_element_type=_f32, precision=lax.Precision.HIGHEST) + cterm
    y1 = jnp.dot(x1, w1t_ref[...], preferred_element_type=_f32, precision=lax.Precision.HIGHEST)
    b1 = y1 + cterm
    y_ref[0] = y1[:, :HALF]
    y_ref[1] = y1[:, HALF:]
    b1_ref[0] = b1[:, :HALF]
    b1_ref[1] = b1[:, HALF:]


_tc_conv = pl.pallas_call(
    _tc_conv_body,
    grid=(NBLK,),
    in_specs=[
        pl.BlockSpec((2, BLK, HALF), lambda i: (0, i, 0)),
        pl.BlockSpec((2, BLK, DE), lambda i: (0, i, 0)),
        pl.BlockSpec((D, D), lambda i: (0, 0)),
        pl.BlockSpec((DE, D), lambda i: (0, 0)),
        pl.BlockSpec((1, D), lambda i: (0, 0)),
    ],
    out_specs=[
        pl.BlockSpec((2, BLK, HALF), lambda i: (0, i, 0)),
        pl.BlockSpec((2, BLK, HALF), lambda i: (0, i, 0)),
    ],
    out_shape=[
        jax.ShapeDtypeStruct((2, NP, HALF), _f32),
        jax.ShapeDtypeStruct((2, NP, HALF), _f32),
    ],
)


def _tc_pool_body(x_ref, batch_ref, wot_ref, b_ref, mf_ref):
    x2 = jnp.concatenate([x_ref[0], x_ref[1]], axis=-1)
    l = jnp.dot(x2, wot_ref[...], preferred_element_type=_f32, precision=lax.Precision.HIGHEST) + b_ref[...]
    m = jnp.max(l, axis=1, keepdims=True)
    p = jnp.exp(l - m)
    a = p / jnp.sum(p, axis=1, keepdims=True)
    bt = batch_ref[...]
    gid = lax.broadcasted_iota(jnp.int32, (1, G), 1)
    oh = (bt == gid).astype(_f32)
    part = lax.dot_general(oh, a, (((0,), (0,)), ((), ())),
                           preferred_element_type=_f32, precision=lax.Precision.HIGHEST)
    i = pl.program_id(0)

    @pl.when(i == 0)
    def _():
        mf_ref[...] = part

    @pl.when(i > 0)
    def _():
        mf_ref[...] += part


_tc_pool = pl.pallas_call(
    _tc_pool_body,
    grid=(NBLK,),
    in_specs=[
        pl.BlockSpec((2, BLK, HALF), lambda i: (0, i, 0)),
        pl.BlockSpec((BLK, 1), lambda i: (i, 0)),
        pl.BlockSpec((D, 512), lambda i: (0, 0)),
        pl.BlockSpec((1, 512), lambda i: (0, 0)),
    ],
    out_specs=pl.BlockSpec((G, 512), lambda i: (0, 0)),
    out_shape=jax.ShapeDtypeStruct((G, 512), _f32),
)


def _tc_readout_body(mf_ref, wh1_ref, bh1_ref, wh2_ref, bh2_ref,
                     wp1_ref, bp1_ref, wp2_ref, bp2_ref, o_ref, z_ref):
    mf = mf_ref[...]
    h1 = jnp.maximum(
        jnp.dot(mf, wh1_ref[...], preferred_element_type=_f32, precision=lax.Precision.HIGHEST) + bh1_ref[...], 0.0)
    z_ref[...] = jnp.dot(h1, wh2_ref[...], preferred_element_type=_f32, precision=lax.Precision.HIGHEST) + bh2_ref[...]
    hid = jnp.dot(mf, wp1_ref[...], preferred_element_type=_f32, precision=lax.Precision.HIGHEST) + bp1_ref[...]
    o = jnp.dot(hid, wp2_ref[...], preferred_element_type=_f32, precision=lax.Precision.HIGHEST) + bp2_ref[...]
    o_ref[...] = jax.nn.sigmoid(o)


_tc_readout = pl.pallas_call(
    _tc_readout_body,
    out_shape=[
        jax.ShapeDtypeStruct((G, 1), _f32),
        jax.ShapeDtypeStruct((G, 1), _f32),
    ],
)


def kernel(x, edge_index, edge_attr, smiles, batch, is_supervised,
           W_in, b_in, W_out, b_out, W_p1, b_p1, W_p2, b_p2,
           W_h1, b_h1, W_h2, b_h2):
    del smiles, is_supervised
    xp = jnp.zeros((NP, D), _f32).at[:N].set(x)
    xh = xp.reshape(NP, 2, HALF).transpose(1, 0, 2)
    pad = EP - E
    srcp = jnp.concatenate(
        [edge_index[0], jnp.zeros((pad,), jnp.int32)]).reshape(16, CHUNKS, 128)
    dstp = jnp.concatenate(
        [edge_index[1], jnp.full((pad,), DUMMY, jnp.int32)]
    ).reshape(16, CHUNKS, 128)
    ea_dst = jnp.concatenate(
        [edge_index[1], jnp.zeros((pad,), jnp.int32)]
    ).reshape(2, 16, EA_CHUNKS, 128)
    ea_val = jnp.zeros((EP, DE), _f32).at[:E].set(edge_attr)
    ea_val = ea_val.reshape(2, 16, EP // 32, DE)
    zeros_ea = jnp.zeros((NP, DE), _f32)
    batchp = jnp.concatenate(
        [batch, jnp.full((NP - N,), G, jnp.int32)]).reshape(NP, 1)
    w1t = W_in[:, :D].T
    w2t = W_in[:, D:].T

    def _emu(table, init, sp, dp):
        outs = []
        for cc in range(2):
            accv = init[cc].at[dp.reshape(-1)].add(table[cc][sp.reshape(-1)])
            outs.append(accv)
        return jnp.stack(outs)
    def _emu_ea(ed, ev):
        outs = []
        for cc in range(2):
            accv = jnp.zeros((NP, DE), _f32).at[ed[cc].reshape(-1)].add(ev[cc].reshape(-1, DE))
            outs.append(accv)
        return jnp.stack(outs)
    s0 = _sc_spmv(False)(lax.optimization_barrier(xh), xh, srcp, dstp)[0]
    eap = _emu_ea(ea_dst, ea_val)
    y1, b1 = _tc_conv(s0, eap, w1t, w2t, b_in.reshape(1, D))
    x2 = _sc_spmv(False)(y1, b1, srcp, dstp)[0]
    mf = _tc_pool(x2, batchp, W_out.T, b_out.reshape(1, 512))
    o, z = _tc_readout(mf, W_h1.T, b_h1.reshape(1, -1), W_h2.T,
                       b_h2.reshape(1, -1), W_p1.T, b_p1.reshape(1, -1),
                       W_p2.T, b_p2.reshape(1, -1))
    return (o, z)


# speedup vs baseline: 4.3965x; 1.4848x over previous
import functools

import jax
import jax.numpy as jnp
from jax import lax
from jax.experimental import pallas as pl
from jax.experimental.pallas import tpu as pltpu
from jax.experimental.pallas import tpu_sc as plsc

N = 10000
E = 160000
D = 256
DE = 16
G = 512
HALF = 128
NP = 10240
STRIPE = NP // 16
DUMMY = N
EP = 163840
CHUNKS = 80
EA_CHUNKS = 40
BLK = 2048
NBLK = NP // BLK

_f32 = jnp.float32


def _make_sc_spmv():
    mesh = plsc.VectorSubcoreMesh(core_axis_name="c", subcore_axis_name="s",
                                  num_cores=2, num_subcores=16)
    HC = CHUNKS // 2
    scratch = [
        pltpu.VMEM_SHARED((NP, HALF), _f32),
        pltpu.VMEM((HC, 128), jnp.int32),
        pltpu.VMEM((HC, 128), jnp.int32),
        pltpu.VMEM((128, HALF), _f32),
        pltpu.VMEM((128, HALF), _f32),
        pltpu.SemaphoreType.DMA,
        pltpu.SemaphoreType.DMA,
    ]

    def body(table, init, srcs, dsts, out, acc, src_v, dst_v, g0, g1, s0, s1):
        c = lax.axis_index("c")
        t = lax.axis_index("s")
        r0 = t * STRIPE

        def init_chunk(h, carry):
            rh = r0 + h * 128
            pltpu.sync_copy(init.at[c, pl.ds(rh, 128)], g0)
            pltpu.sync_copy(g0, acc.at[pl.ds(rh, 128)])
            return carry

        lax.fori_loop(0, STRIPE // 128, init_chunk, 0, unroll=False)
        plsc.subcore_barrier()

        tbl = table.at[c]

        for half in range(2):
            pltpu.sync_copy(srcs.at[t, pl.ds(half * HC, HC)], src_v)
            pltpu.sync_copy(dsts.at[t, pl.ds(half * HC, HC)], dst_v)
            pltpu.async_copy(tbl.at[src_v.at[0]], g0, s0)

            def pipe(q, carry):
                j0 = 2 * q
                j1 = 2 * q + 1
                pltpu.async_copy(tbl.at[src_v.at[j1]], g1, s1)
                pltpu.make_async_copy(tbl.at[src_v.at[j0]], g0, s0).wait()
                pltpu.sync_copy(g0, acc.at[dst_v.at[j0]], add=True)

                @pl.when(q < HC // 2 - 1)
                def _():
                    pltpu.async_copy(tbl.at[src_v.at[j0 + 2]], g0, s0)

                pltpu.make_async_copy(tbl.at[src_v.at[j1]], g1, s1).wait()
                pltpu.sync_copy(g1, acc.at[dst_v.at[j1]], add=True)
                return carry

            lax.fori_loop(0, HC // 2, pipe, 0, unroll=False)

        plsc.subcore_barrier()

        def out_chunk(h, carry):
            rh = r0 + h * 128
            pltpu.sync_copy(acc.at[pl.ds(rh, 128)], g0)
            pltpu.sync_copy(g0, out.at[c, pl.ds(rh, 128)])
            return carry

        lax.fori_loop(0, STRIPE // 128, out_chunk, 0, unroll=False)

    return pl.kernel(body,
                     out_type=(jax.ShapeDtypeStruct((2, NP, HALF), _f32),),
                     mesh=mesh, scratch_types=scratch)


def _make_sc_ea():
    mesh = plsc.VectorSubcoreMesh(core_axis_name="c", subcore_axis_name="s",
                                  num_cores=2, num_subcores=16)
    scratch = [
        pltpu.VMEM_SHARED((NP, HALF), _f32),
        pltpu.VMEM((EA_CHUNKS, 128), jnp.int32),
        pltpu.VMEM((128, HALF), _f32),
    ]

    def body(ea_val, ea_dst, zeros128, out, acc, dst_v, vbuf):
        c = lax.axis_index("c")
        t = lax.axis_index("s")
        r0 = t * STRIPE
        pltpu.sync_copy(ea_dst.at[c, t], dst_v)
        pltpu.sync_copy(zeros128, vbuf)

        def init_chunk(h, carry):
            pltpu.sync_copy(vbuf, acc.at[pl.ds(r0 + h * 128, 128)])
            return carry

        lax.fori_loop(0, STRIPE // 128, init_chunk, 0, unroll=False)
        plsc.subcore_barrier()

        def ea_chunk(j, carry):
            pltpu.sync_copy(ea_val.at[c, t, pl.ds(j * 128, 128)], vbuf)
            pltpu.sync_copy(vbuf, acc.at[dst_v.at[j]], add=True)
            return carry

        lax.fori_loop(0, EA_CHUNKS, ea_chunk, 0, unroll=False)
        plsc.subcore_barrier()

        def out_chunk(h, carry):
            rh = r0 + h * 128
            pltpu.sync_copy(acc.at[pl.ds(rh, 128)], vbuf)
            pltpu.sync_copy(vbuf, out.at[c, pl.ds(rh, 128)])
            return carry

        lax.fori_loop(0, STRIPE // 128, out_chunk, 0, unroll=False)

    return pl.kernel(body,
                     out_type=(jax.ShapeDtypeStruct((2, NP, HALF), _f32),),
                     mesh=mesh, scratch_types=scratch)


@functools.lru_cache(maxsize=None)
def _sc_spmv():
    return _make_sc_spmv()


@functools.lru_cache(maxsize=None)
def _sc_ea():
    return _make_sc_ea()


def _tc_conv_body(s_ref, ea_ref, w1t_ref, w2t_ref, b_ref, y_ref, b1_ref):
    s = jnp.concatenate([s_ref[0], s_ref[1]], axis=-1)
    ea = ea_ref[0] + ea_ref[1]
    cterm = jnp.dot(ea, w2t_ref[...], preferred_element_type=_f32, precision=lax.Precision.HIGHEST) + b_ref[...]
    x1 = jnp.dot(s, w1t_ref[...], preferred_element_type=_f32, precision=lax.Precision.HIGHEST) + cterm
    y1 = jnp.dot(x1, w1t_ref[...], preferred_element_type=_f32, precision=lax.Precision.HIGHEST)
    b1 = y1 + cterm
    y_ref[0] = y1[:, :HALF]
    y_ref[1] = y1[:, HALF:]
    b1_ref[0] = b1[:, :HALF]
    b1_ref[1] = b1[:, HALF:]


_tc_conv = pl.pallas_call(
    _tc_conv_body,
    grid=(NBLK,),
    in_specs=[
        pl.BlockSpec((2, BLK, HALF), lambda i: (0, i, 0)),
        pl.BlockSpec((2, BLK, DE), lambda i: (0, i, 0)),
        pl.BlockSpec((D, D), lambda i: (0, 0)),
        pl.BlockSpec((DE, D), lambda i: (0, 0)),
        pl.BlockSpec((1, D), lambda i: (0, 0)),
    ],
    out_specs=[
        pl.BlockSpec((2, BLK, HALF), lambda i: (0, i, 0)),
        pl.BlockSpec((2, BLK, HALF), lambda i: (0, i, 0)),
    ],
    out_shape=[
        jax.ShapeDtypeStruct((2, NP, HALF), _f32),
        jax.ShapeDtypeStruct((2, NP, HALF), _f32),
    ],
)


def _tc_pool_body(x_ref, batch_ref, wot_ref, b_ref, mf_ref):
    x2 = jnp.concatenate([x_ref[0], x_ref[1]], axis=-1)
    l = jnp.dot(x2, wot_ref[...], preferred_element_type=_f32, precision=lax.Precision.HIGHEST) + b_ref[...]
    m = jnp.max(l, axis=1, keepdims=True)
    p = jnp.exp(l - m)
    a = p / jnp.sum(p, axis=1, keepdims=True)
    bt = batch_ref[...]
    gid = lax.broadcasted_iota(jnp.int32, (1, G), 1)
    oh = (bt == gid).astype(_f32)
    part = lax.dot_general(oh, a, (((0,), (0,)), ((), ())),
                           preferred_element_type=_f32, precision=lax.Precision.HIGHEST)
    i = pl.program_id(0)

    @pl.when(i == 0)
    def _():
        mf_ref[...] = part

    @pl.when(i > 0)
    def _():
        mf_ref[...] += part


_tc_pool = pl.pallas_call(
    _tc_pool_body,
    grid=(NBLK,),
    in_specs=[
        pl.BlockSpec((2, BLK, HALF), lambda i: (0, i, 0)),
        pl.BlockSpec((BLK, 1), lambda i: (i, 0)),
        pl.BlockSpec((D, 512), lambda i: (0, 0)),
        pl.BlockSpec((1, 512), lambda i: (0, 0)),
    ],
    out_specs=pl.BlockSpec((G, 512), lambda i: (0, 0)),
    out_shape=jax.ShapeDtypeStruct((G, 512), _f32),
)


def _tc_readout_body(mf_ref, wh1_ref, bh1_ref, wh2_ref, bh2_ref,
                     wp1_ref, bp1_ref, wp2_ref, bp2_ref, o_ref, z_ref):
    mf = mf_ref[...]
    h1 = jnp.maximum(
        jnp.dot(mf, wh1_ref[...], preferred_element_type=_f32, precision=lax.Precision.HIGHEST) + bh1_ref[...], 0.0)
    z_ref[...] = jnp.dot(h1, wh2_ref[...], preferred_element_type=_f32, precision=lax.Precision.HIGHEST) + bh2_ref[...]
    hid = jnp.dot(mf, wp1_ref[...], preferred_element_type=_f32, precision=lax.Precision.HIGHEST) + bp1_ref[...]
    o = jnp.dot(hid, wp2_ref[...], preferred_element_type=_f32, precision=lax.Precision.HIGHEST) + bp2_ref[...]
    o_ref[...] = jax.nn.sigmoid(o)


_tc_readout = pl.pallas_call(
    _tc_readout_body,
    out_shape=[
        jax.ShapeDtypeStruct((G, 1), _f32),
        jax.ShapeDtypeStruct((G, 1), _f32),
    ],
)


def kernel(x, edge_index, edge_attr, smiles, batch, is_supervised,
           W_in, b_in, W_out, b_out, W_p1, b_p1, W_p2, b_p2,
           W_h1, b_h1, W_h2, b_h2):
    del smiles, is_supervised
    xp = jnp.zeros((NP, D), _f32).at[:N].set(x)
    xh = xp.reshape(NP, 2, HALF).transpose(1, 0, 2)
    pad = EP - E
    srcp = jnp.concatenate(
        [edge_index[0], jnp.zeros((pad,), jnp.int32)]).reshape(16, CHUNKS, 128)
    dstp = jnp.concatenate(
        [edge_index[1], jnp.full((pad,), DUMMY, jnp.int32)]
    ).reshape(16, CHUNKS, 128)
    ea_dst = jnp.concatenate(
        [edge_index[1], jnp.zeros((pad,), jnp.int32)]
    ).reshape(2, 16, EA_CHUNKS, 128)
    ea128 = jnp.zeros((EP, HALF), _f32).at[:E, :DE].set(edge_attr)
    ea128 = ea128.reshape(2, 16, EP // 32, HALF)
    zeros128 = jnp.zeros((128, HALF), _f32)
    batchp = jnp.concatenate(
        [batch, jnp.full((NP - N,), G, jnp.int32)]).reshape(NP, 1)
    w1t = W_in[:, :D].T
    w2t = W_in[:, D:].T

    eap_full = _sc_ea()(ea128, ea_dst, zeros128)[0]
    eap = eap_full[:, :, :DE]
    s0 = _sc_spmv()(lax.optimization_barrier(xh), xh, srcp, dstp)[0]
    y1, b1 = _tc_conv(s0, eap, w1t, w2t, b_in.reshape(1, D))
    x2 = _sc_spmv()(y1, b1, srcp, dstp)[0]
    mf = _tc_pool(x2, batchp, W_out.T, b_out.reshape(1, 512))
    o, z = _tc_readout(mf, W_h1.T, b_h1.reshape(1, -1), W_h2.T,
                       b_h2.reshape(1, -1), W_p1.T, b_p1.reshape(1, -1),
                       W_p2.T, b_p2.reshape(1, -1))
    return (o, z)


# speedup vs baseline: 4.4708x; 1.0169x over previous
import functools

import jax
import jax.numpy as jnp
from jax import lax
from jax.experimental import pallas as pl
from jax.experimental.pallas import tpu as pltpu
from jax.experimental.pallas import tpu_sc as plsc

N = 10000
E = 160000
D = 256
DE = 16
G = 512
HALF = 128
NP = 10240
STRIPE = NP // 16
DUMMY = N
EP = 163840
CHUNKS = 80
EA_CHUNKS = 40
BLK = 2048
NBLK = NP // BLK

_f32 = jnp.float32


def _make_sc_spmv():
    mesh = plsc.VectorSubcoreMesh(core_axis_name="c", subcore_axis_name="s",
                                  num_cores=2, num_subcores=16)
    HC = CHUNKS // 2
    scratch = [
        pltpu.VMEM_SHARED((NP, HALF), _f32),
        pltpu.VMEM((HC, 128), jnp.int32),
        pltpu.VMEM((HC, 128), jnp.int32),
        pltpu.VMEM((128, HALF), _f32),
        pltpu.VMEM((128, HALF), _f32),
        pltpu.SemaphoreType.DMA,
        pltpu.SemaphoreType.DMA,
        pltpu.SemaphoreType.DMA,
    ]

    def body(table, init, srcs, dsts, out, acc, src_v, dst_v, g0, g1,
             s0, s1, ss):
        c = lax.axis_index("c")
        t = lax.axis_index("s")
        r0 = t * STRIPE

        def init_chunk(h, carry):
            rh = r0 + h * 128
            pltpu.sync_copy(init.at[c, pl.ds(rh, 128)], g0)
            pltpu.sync_copy(g0, acc.at[pl.ds(rh, 128)])
            return carry

        lax.fori_loop(0, STRIPE // 128, init_chunk, 0, unroll=False)
        plsc.subcore_barrier()

        tbl = table.at[c]

        for half in range(2):
            pltpu.sync_copy(srcs.at[t, pl.ds(half * HC, HC)], src_v)
            pltpu.sync_copy(dsts.at[t, pl.ds(half * HC, HC)], dst_v)
            pltpu.async_copy(tbl.at[src_v.at[0]], g0, s0)
            pltpu.async_copy(tbl.at[src_v.at[1]], g1, s1)

            def pipe(q, carry):
                j0 = 2 * q
                j1 = 2 * q + 1
                pltpu.make_async_copy(tbl.at[src_v.at[j0]], g0, s0).wait()
                d0 = pltpu.async_copy(g0, acc.at[dst_v.at[j0]], ss, add=True)
                pltpu.make_async_copy(tbl.at[src_v.at[j1]], g1, s1).wait()
                d1 = pltpu.async_copy(g1, acc.at[dst_v.at[j1]], ss, add=True)
                d0.wait()

                @pl.when(q < HC // 2 - 1)
                def _():
                    pltpu.async_copy(tbl.at[src_v.at[j0 + 2]], g0, s0)

                d1.wait()

                @pl.when(q < HC // 2 - 1)
                def _():
                    pltpu.async_copy(tbl.at[src_v.at[j1 + 2]], g1, s1)

                return carry

            lax.fori_loop(0, HC // 2, pipe, 0, unroll=False)

        plsc.subcore_barrier()

        def out_chunk(h, carry):
            rh = r0 + h * 128
            pltpu.sync_copy(acc.at[pl.ds(rh, 128)], g0)
            pltpu.sync_copy(g0, out.at[c, pl.ds(rh, 128)])
            return carry

        lax.fori_loop(0, STRIPE // 128, out_chunk, 0, unroll=False)

    return pl.kernel(body,
                     out_type=(jax.ShapeDtypeStruct((2, NP, HALF), _f32),),
                     mesh=mesh, scratch_types=scratch)


def _make_sc_ea():
    mesh = plsc.VectorSubcoreMesh(core_axis_name="c", subcore_axis_name="s",
                                  num_cores=2, num_subcores=16)
    scratch = [
        pltpu.VMEM_SHARED((NP, HALF), _f32),
        pltpu.VMEM((EA_CHUNKS, 128), jnp.int32),
        pltpu.VMEM((128, HALF), _f32),
        pltpu.VMEM((128, HALF), _f32),
        pltpu.SemaphoreType.DMA,
        pltpu.SemaphoreType.DMA,
        pltpu.SemaphoreType.DMA,
    ]

    def body(ea_val, ea_dst, zeros128, out, acc, dst_v, vbuf, vbuf1,
             s0, s1, ss):
        c = lax.axis_index("c")
        t = lax.axis_index("s")
        r0 = t * STRIPE
        pltpu.sync_copy(ea_dst.at[c, t], dst_v)
        pltpu.sync_copy(zeros128, vbuf)

        def init_chunk(h, carry):
            pltpu.sync_copy(vbuf, acc.at[pl.ds(r0 + h * 128, 128)])
            return carry

        lax.fori_loop(0, STRIPE // 128, init_chunk, 0, unroll=False)
        plsc.subcore_barrier()

        pltpu.async_copy(ea_val.at[c, t, pl.ds(0, 128)], vbuf, s0)
        pltpu.async_copy(ea_val.at[c, t, pl.ds(128, 128)], vbuf1, s1)

        def ea_pipe(q, carry):
            j0 = 2 * q
            j1 = 2 * q + 1
            pltpu.make_async_copy(
                ea_val.at[c, t, pl.ds(j0 * 128, 128)], vbuf, s0).wait()
            d0 = pltpu.async_copy(vbuf, acc.at[dst_v.at[j0]], ss, add=True)
            pltpu.make_async_copy(
                ea_val.at[c, t, pl.ds(j1 * 128, 128)], vbuf1, s1).wait()
            d1 = pltpu.async_copy(vbuf1, acc.at[dst_v.at[j1]], ss, add=True)
            d0.wait()

            @pl.when(q < EA_CHUNKS // 2 - 1)
            def _():
                pltpu.async_copy(
                    ea_val.at[c, t, pl.ds((j0 + 2) * 128, 128)], vbuf, s0)

            d1.wait()

            @pl.when(q < EA_CHUNKS // 2 - 1)
            def _():
                pltpu.async_copy(
                    ea_val.at[c, t, pl.ds((j1 + 2) * 128, 128)], vbuf1, s1)

            return carry

        lax.fori_loop(0, EA_CHUNKS // 2, ea_pipe, 0, unroll=False)
        plsc.subcore_barrier()

        def out_chunk(h, carry):
            rh = r0 + h * 128
            pltpu.sync_copy(acc.at[pl.ds(rh, 128)], vbuf)
            pltpu.sync_copy(vbuf, out.at[c, pl.ds(rh, 128)])
            return carry

        lax.fori_loop(0, STRIPE // 128, out_chunk, 0, unroll=False)

    return pl.kernel(body,
                     out_type=(jax.ShapeDtypeStruct((2, NP, HALF), _f32),),
                     mesh=mesh, scratch_types=scratch)


@functools.lru_cache(maxsize=None)
def _sc_spmv():
    return _make_sc_spmv()


@functools.lru_cache(maxsize=None)
def _sc_ea():
    return _make_sc_ea()


def _tc_conv_body(s_ref, ea_ref, w1t_ref, w2t_ref, b_ref, y_ref, b1_ref):
    s = jnp.concatenate([s_ref[0], s_ref[1]], axis=-1)
    ea = ea_ref[0] + ea_ref[1]
    cterm = jnp.dot(ea, w2t_ref[...], preferred_element_type=_f32, precision=lax.Precision.HIGHEST) + b_ref[...]
    x1 = jnp.dot(s, w1t_ref[...], preferred_element_type=_f32, precision=lax.Precision.HIGHEST) + cterm
    y1 = jnp.dot(x1, w1t_ref[...], preferred_element_type=_f32, precision=lax.Precision.HIGHEST)
    b1 = y1 + cterm
    y_ref[0] = y1[:, :HALF]
    y_ref[1] = y1[:, HALF:]
    b1_ref[0] = b1[:, :HALF]
    b1_ref[1] = b1[:, HALF:]


_tc_conv = pl.pallas_call(
    _tc_conv_body,
    grid=(NBLK,),
    in_specs=[
        pl.BlockSpec((2, BLK, HALF), lambda i: (0, i, 0)),
        pl.BlockSpec((2, BLK, DE), lambda i: (0, i, 0)),
        pl.BlockSpec((D, D), lambda i: (0, 0)),
        pl.BlockSpec((DE, D), lambda i: (0, 0)),
        pl.BlockSpec((1, D), lambda i: (0, 0)),
    ],
    out_specs=[
        pl.BlockSpec((2, BLK, HALF), lambda i: (0, i, 0)),
        pl.BlockSpec((2, BLK, HALF), lambda i: (0, i, 0)),
    ],
    out_shape=[
        jax.ShapeDtypeStruct((2, NP, HALF), _f32),
        jax.ShapeDtypeStruct((2, NP, HALF), _f32),
    ],
)


def _tc_pool_body(x_ref, batch_ref, wot_ref, b_ref, mf_ref):
    x2 = jnp.concatenate([x_ref[0], x_ref[1]], axis=-1)
    l = jnp.dot(x2, wot_ref[...], preferred_element_type=_f32, precision=lax.Precision.HIGHEST) + b_ref[...]
    m = jnp.max(l, axis=1, keepdims=True)
    p = jnp.exp(l - m)
    a = p / jnp.sum(p, axis=1, keepdims=True)
    bt = batch_ref[...]
    gid = lax.broadcasted_iota(jnp.int32, (1, G), 1)
    oh = (bt == gid).astype(_f32)
    part = lax.dot_general(oh, a, (((0,), (0,)), ((), ())),
                           preferred_element_type=_f32, precision=lax.Precision.HIGHEST)
    i = pl.program_id(0)

    @pl.when(i == 0)
    def _():
        mf_ref[...] = part

    @pl.when(i > 0)
    def _():
        mf_ref[...] += part


_tc_pool = pl.pallas_call(
    _tc_pool_body,
    grid=(NBLK,),
    in_specs=[
        pl.BlockSpec((2, BLK, HALF), lambda i: (0, i, 0)),
        pl.BlockSpec((BLK, 1), lambda i: (i, 0)),
        pl.BlockSpec((D, 512), lambda i: (0, 0)),
        pl.BlockSpec((1, 512), lambda i: (0, 0)),
    ],
    out_specs=pl.BlockSpec((G, 512), lambda i: (0, 0)),
    out_shape=jax.ShapeDtypeStruct((G, 512), _f32),
)


def _tc_readout_body(mf_ref, wh1_ref, bh1_ref, wh2_ref, bh2_ref,
                     wp1_ref, bp1_ref, wp2_ref, bp2_ref, o_ref, z_ref):
    mf = mf_ref[...]
    h1 = jnp.maximum(
        jnp.dot(mf, wh1_ref[...], preferred_element_type=_f32, precision=lax.Precision.HIGHEST) + bh1_ref[...], 0.0)
    z_ref[...] = jnp.dot(h1, wh2_ref[...], preferred_element_type=_f32, precision=lax.Precision.HIGHEST) + bh2_ref[...]
    hid = jnp.dot(mf, wp1_ref[...], preferred_element_type=_f32, precision=lax.Precision.HIGHEST) + bp1_ref[...]
    o = jnp.dot(hid, wp2_ref[...], preferred_element_type=_f32, precision=lax.Precision.HIGHEST) + bp2_ref[...]
    o_ref[...] = jax.nn.sigmoid(o)


_tc_readout = pl.pallas_call(
    _tc_readout_body,
    out_shape=[
        jax.ShapeDtypeStruct((G, 1), _f32),
        jax.ShapeDtypeStruct((G, 1), _f32),
    ],
)


def kernel(x, edge_index, edge_attr, smiles, batch, is_supervised,
           W_in, b_in, W_out, b_out, W_p1, b_p1, W_p2, b_p2,
           W_h1, b_h1, W_h2, b_h2):
    del smiles, is_supervised
    xp = jnp.zeros((NP, D), _f32).at[:N].set(x)
    xh = xp.reshape(NP, 2, HALF).transpose(1, 0, 2)
    pad = EP - E
    srcp = jnp.concatenate(
        [edge_index[0], jnp.zeros((pad,), jnp.int32)]).reshape(16, CHUNKS, 128)
    dstp = jnp.concatenate(
        [edge_index[1], jnp.full((pad,), DUMMY, jnp.int32)]
    ).reshape(16, CHUNKS, 128)
    ea_dst = jnp.concatenate(
        [edge_index[1], jnp.zeros((pad,), jnp.int32)]
    ).reshape(2, 16, EA_CHUNKS, 128)
    ea128 = jnp.zeros((EP, HALF), _f32).at[:E, :DE].set(edge_attr)
    ea128 = ea128.reshape(2, 16, EP // 32, HALF)
    zeros128 = jnp.zeros((128, HALF), _f32)
    batchp = jnp.concatenate(
        [batch, jnp.full((NP - N,), G, jnp.int32)]).reshape(NP, 1)
    w1t = W_in[:, :D].T
    w2t = W_in[:, D:].T

    eap_full = _sc_ea()(ea128, ea_dst, zeros128)[0]
    eap = eap_full[:, :, :DE]
    s0 = _sc_spmv()(lax.optimization_barrier(xh), xh, srcp, dstp)[0]
    y1, b1 = _tc_conv(s0, eap, w1t, w2t, b_in.reshape(1, D))
    x2 = _sc_spmv()(y1, b1, srcp, dstp)[0]
    mf = _tc_pool(x2, batchp, W_out.T, b_out.reshape(1, 512))
    o, z = _tc_readout(mf, W_h1.T, b_h1.reshape(1, -1), W_h2.T,
                       b_h2.reshape(1, -1), W_p1.T, b_p1.reshape(1, -1),
                       W_p2.T, b_p2.reshape(1, -1))
    return (o, z)


# speedup vs baseline: 4.6179x; 1.0329x over previous
import functools

import jax
import jax.numpy as jnp
from jax import lax
from jax.experimental import pallas as pl
from jax.experimental.pallas import tpu as pltpu
from jax.experimental.pallas import tpu_sc as plsc

N = 10000
E = 160000
D = 256
DE = 16
G = 512
HALF = 128
NP = 10240
STRIPE = NP // 16
DUMMY = N
EP = 163840
CHUNKS = 80
EA_CHUNKS = 40
BLK = 2048
NBLK = NP // BLK

_f32 = jnp.float32


def _make_sc_spmv():
    mesh = plsc.VectorSubcoreMesh(core_axis_name="c", subcore_axis_name="s",
                                  num_cores=2, num_subcores=16)
    HC = CHUNKS // 2
    scratch = [
        pltpu.VMEM_SHARED((NP, HALF), _f32),
        pltpu.VMEM((HC, 128), jnp.int32),
        pltpu.VMEM((HC, 128), jnp.int32),
        pltpu.VMEM((128, HALF), _f32),
        pltpu.VMEM((128, HALF), _f32),
        pltpu.SemaphoreType.DMA,
        pltpu.SemaphoreType.DMA,
        pltpu.SemaphoreType.DMA,
    ]

    def body(table, init, srcs, dsts, out, acc, src_v, dst_v, g0, g1,
             s0, s1, ss):
        c = lax.axis_index("c")
        t = lax.axis_index("s")
        r0 = t * STRIPE

        def init_chunk(h, carry):
            rh = r0 + h * 128
            pltpu.sync_copy(init.at[c, pl.ds(rh, 128)], g0)
            pltpu.sync_copy(g0, acc.at[pl.ds(rh, 128)])
            return carry

        lax.fori_loop(0, STRIPE // 128, init_chunk, 0, unroll=False)
        plsc.subcore_barrier()

        tbl = table.at[c]

        for half in range(2):
            pltpu.sync_copy(srcs.at[t, pl.ds(half * HC, HC)], src_v)
            pltpu.sync_copy(dsts.at[t, pl.ds(half * HC, HC)], dst_v)
            pltpu.async_copy(tbl.at[src_v.at[0]], g0, s0)
            pltpu.async_copy(tbl.at[src_v.at[1]], g1, s1)

            def pipe(q, carry):
                j0 = 2 * q
                j1 = 2 * q + 1
                pltpu.make_async_copy(tbl.at[src_v.at[j0]], g0, s0).wait()
                d0 = pltpu.async_copy(g0, acc.at[dst_v.at[j0]], ss, add=True)
                pltpu.make_async_copy(tbl.at[src_v.at[j1]], g1, s1).wait()
                d1 = pltpu.async_copy(g1, acc.at[dst_v.at[j1]], ss, add=True)
                d0.wait()

                @pl.when(q < HC // 2 - 1)
                def _():
                    pltpu.async_copy(tbl.at[src_v.at[j0 + 2]], g0, s0)

                d1.wait()

                @pl.when(q < HC // 2 - 1)
                def _():
                    pltpu.async_copy(tbl.at[src_v.at[j1 + 2]], g1, s1)

                return carry

            lax.fori_loop(0, HC // 2, pipe, 0, unroll=False)

        plsc.subcore_barrier()

        def out_chunk(h, carry):
            rh = r0 + h * 128
            pltpu.sync_copy(acc.at[pl.ds(rh, 128)], g0)
            pltpu.sync_copy(g0, out.at[c, pl.ds(rh, 128)])
            return carry

        lax.fori_loop(0, STRIPE // 128, out_chunk, 0, unroll=False)

    return pl.kernel(body,
                     out_type=(jax.ShapeDtypeStruct((2, NP, HALF), _f32),),
                     mesh=mesh, scratch_types=scratch)


def _make_sc_ea():
    mesh = plsc.VectorSubcoreMesh(core_axis_name="c", subcore_axis_name="s",
                                  num_cores=2, num_subcores=16)
    scratch = [
        pltpu.VMEM_SHARED((NP, HALF), _f32),
        pltpu.VMEM((EA_CHUNKS, 128), jnp.int32),
        pltpu.VMEM((128, HALF), _f32),
        pltpu.VMEM((128, HALF), _f32),
        pltpu.SemaphoreType.DMA,
        pltpu.SemaphoreType.DMA,
        pltpu.SemaphoreType.DMA,
    ]

    def body(ea_val, ea_dst, zeros128, out, acc, dst_v, vbuf, vbuf1,
             s0, s1, ss):
        c = lax.axis_index("c")
        t = lax.axis_index("s")
        r0 = t * STRIPE
        pltpu.sync_copy(ea_dst.at[c, t], dst_v)
        pltpu.sync_copy(zeros128, vbuf)

        def init_chunk(h, carry):
            pltpu.sync_copy(vbuf, acc.at[pl.ds(r0 + h * 128, 128)])
            return carry

        lax.fori_loop(0, STRIPE // 128, init_chunk, 0, unroll=False)
        plsc.subcore_barrier()

        pltpu.async_copy(ea_val.at[c, t, pl.ds(0, 128)], vbuf, s0)
        pltpu.async_copy(ea_val.at[c, t, pl.ds(128, 128)], vbuf1, s1)

        def ea_pipe(q, carry):
            j0 = 2 * q
            j1 = 2 * q + 1
            pltpu.make_async_copy(
                ea_val.at[c, t, pl.ds(j0 * 128, 128)], vbuf, s0).wait()
            d0 = pltpu.async_copy(vbuf, acc.at[dst_v.at[j0]], ss, add=True)
            pltpu.make_async_copy(
                ea_val.at[c, t, pl.ds(j1 * 128, 128)], vbuf1, s1).wait()
            d1 = pltpu.async_copy(vbuf1, acc.at[dst_v.at[j1]], ss, add=True)
            d0.wait()

            @pl.when(q < EA_CHUNKS // 2 - 1)
            def _():
                pltpu.async_copy(
                    ea_val.at[c, t, pl.ds((j0 + 2) * 128, 128)], vbuf, s0)

            d1.wait()

            @pl.when(q < EA_CHUNKS // 2 - 1)
            def _():
                pltpu.async_copy(
                    ea_val.at[c, t, pl.ds((j1 + 2) * 128, 128)], vbuf1, s1)

            return carry

        lax.fori_loop(0, EA_CHUNKS // 2, ea_pipe, 0, unroll=False)
        plsc.subcore_barrier()

        def out_chunk(h, carry):
            rh = r0 + h * 128
            pltpu.sync_copy(acc.at[pl.ds(rh, 128)], vbuf)
            pltpu.sync_copy(vbuf, out.at[c, pl.ds(rh, 128)])
            return carry

        lax.fori_loop(0, STRIPE // 128, out_chunk, 0, unroll=False)

    return pl.kernel(body,
                     out_type=(jax.ShapeDtypeStruct((2, NP, HALF), _f32),),
                     mesh=mesh, scratch_types=scratch)


@functools.lru_cache(maxsize=None)
def _sc_spmv():
    return _make_sc_spmv()


@functools.lru_cache(maxsize=None)
def _sc_ea():
    return _make_sc_ea()


def _tc_pre_body(x_ref, ea_ref, w1t_ref, w2t_ref, b_ref, y_ref, b0_ref, ch_ref):
    ea = ea_ref[0] + ea_ref[1]
    ct = jnp.dot(ea, w2t_ref[...], preferred_element_type=_f32,
                 precision=lax.Precision.HIGHEST) + b_ref[...]
    y0 = jnp.dot(x_ref[...], w1t_ref[...], preferred_element_type=_f32,
                 precision=lax.Precision.HIGHEST)
    b0 = y0 + ct
    y_ref[0] = y0[:, :HALF]
    y_ref[1] = y0[:, HALF:]
    b0_ref[0] = b0[:, :HALF]
    b0_ref[1] = b0[:, HALF:]
    ch_ref[0] = ct[:, :HALF]
    ch_ref[1] = ct[:, HALF:]


_tc_pre = pl.pallas_call(
    _tc_pre_body,
    grid=(NBLK,),
    in_specs=[
        pl.BlockSpec((BLK, D), lambda i: (i, 0)),
        pl.BlockSpec((2, BLK, DE), lambda i: (0, i, 0)),
        pl.BlockSpec((D, D), lambda i: (0, 0)),
        pl.BlockSpec((DE, D), lambda i: (0, 0)),
        pl.BlockSpec((1, D), lambda i: (0, 0)),
    ],
    out_specs=[
        pl.BlockSpec((2, BLK, HALF), lambda i: (0, i, 0)),
        pl.BlockSpec((2, BLK, HALF), lambda i: (0, i, 0)),
        pl.BlockSpec((2, BLK, HALF), lambda i: (0, i, 0)),
    ],
    out_shape=[
        jax.ShapeDtypeStruct((2, NP, HALF), _f32),
        jax.ShapeDtypeStruct((2, NP, HALF), _f32),
        jax.ShapeDtypeStruct((2, NP, HALF), _f32),
    ],
)


def _tc_conv_body(x1_ref, ch_ref, w1t_ref, y_ref, b1_ref):
    x1 = jnp.concatenate([x1_ref[0], x1_ref[1]], axis=-1)
    y1 = jnp.dot(x1, w1t_ref[...], preferred_element_type=_f32,
                 precision=lax.Precision.HIGHEST)
    y_ref[0] = y1[:, :HALF]
    y_ref[1] = y1[:, HALF:]
    b1_ref[0] = y1[:, :HALF] + ch_ref[0]
    b1_ref[1] = y1[:, HALF:] + ch_ref[1]


_tc_conv = pl.pallas_call(
    _tc_conv_body,
    grid=(NBLK,),
    in_specs=[
        pl.BlockSpec((2, BLK, HALF), lambda i: (0, i, 0)),
        pl.BlockSpec((2, BLK, HALF), lambda i: (0, i, 0)),
        pl.BlockSpec((D, D), lambda i: (0, 0)),
    ],
    out_specs=[
        pl.BlockSpec((2, BLK, HALF), lambda i: (0, i, 0)),
        pl.BlockSpec((2, BLK, HALF), lambda i: (0, i, 0)),
    ],
    out_shape=[
        jax.ShapeDtypeStruct((2, NP, HALF), _f32),
        jax.ShapeDtypeStruct((2, NP, HALF), _f32),
    ],
)


def _tc_pool_body(x_ref, batch_ref, wot_ref, b_ref,
                  wh1_ref, bh1_ref, wh2_ref, bh2_ref,
                  wp1_ref, bp1_ref, wp2_ref, bp2_ref,
                  mf_ref, o_ref, z_ref):
    x2 = jnp.concatenate([x_ref[0], x_ref[1]], axis=-1)
    l = jnp.dot(x2, wot_ref[...], preferred_element_type=_f32,
                precision=lax.Precision.HIGHEST) + b_ref[...]
    m = jnp.max(l, axis=1, keepdims=True)
    p = jnp.exp(l - m)
    a = p / jnp.sum(p, axis=1, keepdims=True)
    bt = batch_ref[...]
    gid = lax.broadcasted_iota(jnp.int32, (1, G), 1)
    oh = (bt == gid).astype(_f32)
    part = lax.dot_general(oh, a, (((0,), (0,)), ((), ())),
                           preferred_element_type=_f32,
                           precision=lax.Precision.HIGHEST)
    i = pl.program_id(0)

    @pl.when(i == 0)
    def _():
        mf_ref[...] = part

    @pl.when(i > 0)
    def _():
        mf_ref[...] += part

    @pl.when(i == NBLK - 1)
    def _():
        mf = mf_ref[...]
        h1 = jnp.maximum(
            jnp.dot(mf, wh1_ref[...], preferred_element_type=_f32,
                    precision=lax.Precision.HIGHEST) + bh1_ref[...], 0.0)
        z_ref[...] = jnp.dot(h1, wh2_ref[...], preferred_element_type=_f32,
                             precision=lax.Precision.HIGHEST) + bh2_ref[...]
        hid = jnp.dot(mf, wp1_ref[...], preferred_element_type=_f32,
                      precision=lax.Precision.HIGHEST) + bp1_ref[...]
        o = jnp.dot(hid, wp2_ref[...], preferred_element_type=_f32,
                    precision=lax.Precision.HIGHEST) + bp2_ref[...]
        o_ref[...] = jax.nn.sigmoid(o)


_tc_pool = pl.pallas_call(
    _tc_pool_body,
    grid=(NBLK,),
    in_specs=[
        pl.BlockSpec((2, BLK, HALF), lambda i: (0, i, 0)),
        pl.BlockSpec((BLK, 1), lambda i: (i, 0)),
        pl.BlockSpec((D, 512), lambda i: (0, 0)),
        pl.BlockSpec((1, 512), lambda i: (0, 0)),
        pl.BlockSpec((512, 128), lambda i: (0, 0)),
        pl.BlockSpec((1, 128), lambda i: (0, 0)),
        pl.BlockSpec((128, 1), lambda i: (0, 0)),
        pl.BlockSpec((1, 1), lambda i: (0, 0)),
        pl.BlockSpec((512, 50), lambda i: (0, 0)),
        pl.BlockSpec((1, 50), lambda i: (0, 0)),
        pl.BlockSpec((50, 1), lambda i: (0, 0)),
        pl.BlockSpec((1, 1), lambda i: (0, 0)),
    ],
    out_specs=[
        pl.BlockSpec((G, 512), lambda i: (0, 0)),
        pl.BlockSpec((G, 1), lambda i: (0, 0)),
        pl.BlockSpec((G, 1), lambda i: (0, 0)),
    ],
    out_shape=[
        jax.ShapeDtypeStruct((G, 512), _f32),
        jax.ShapeDtypeStruct((G, 1), _f32),
        jax.ShapeDtypeStruct((G, 1), _f32),
    ],
)


def kernel(x, edge_index, edge_attr, smiles, batch, is_supervised,
           W_in, b_in, W_out, b_out, W_p1, b_p1, W_p2, b_p2,
           W_h1, b_h1, W_h2, b_h2):
    del smiles, is_supervised
    xp = jnp.zeros((NP, D), _f32).at[:N].set(x)
    pad = EP - E
    srcp = jnp.concatenate(
        [edge_index[0], jnp.zeros((pad,), jnp.int32)]).reshape(16, CHUNKS, 128)
    dstp = jnp.concatenate(
        [edge_index[1], jnp.full((pad,), DUMMY, jnp.int32)]
    ).reshape(16, CHUNKS, 128)
    ea_dst = jnp.concatenate(
        [edge_index[1], jnp.zeros((pad,), jnp.int32)]
    ).reshape(2, 16, EA_CHUNKS, 128)
    ea128 = jnp.zeros((EP, HALF), _f32).at[:E, :DE].set(edge_attr)
    ea128 = ea128.reshape(2, 16, EP // 32, HALF)
    zeros128 = jnp.zeros((128, HALF), _f32)
    batchp = jnp.concatenate(
        [batch, jnp.full((NP - N,), G, jnp.int32)]).reshape(NP, 1)
    w1t = W_in[:, :D].T
    w2t = W_in[:, D:].T

    eap = _sc_ea()(ea128, ea_dst, zeros128)[0][:, :, :DE]
    y0, b0, ch = _tc_pre(xp, eap, w1t, w2t, b_in.reshape(1, D))
    x1 = _sc_spmv()(y0, b0, srcp, dstp)[0]
    y1, b1 = _tc_conv(x1, ch, w1t)
    x2 = _sc_spmv()(y1, b1, srcp, dstp)[0]
    _, o, z = _tc_pool(x2, batchp, W_out.T, b_out.reshape(1, 512),
                       W_h1.T, b_h1.reshape(1, -1), W_h2.T,
                       b_h2.reshape(1, -1), W_p1.T, b_p1.reshape(1, -1),
                       W_p2.T, b_p2.reshape(1, -1))
    return (o, z)


# speedup vs baseline: 4.6753x; 1.0124x over previous
import functools

import jax
import jax.numpy as jnp
from jax import lax
from jax.experimental import pallas as pl
from jax.experimental.pallas import tpu as pltpu
from jax.experimental.pallas import tpu_sc as plsc

N = 10000
E = 160000
D = 256
DE = 16
G = 512
HALF = 128
NP = 10240
STRIPE = NP // 16
DUMMY = N
EP = 163840
CHUNKS = 80
EA_CHUNKS = 40
BLK = 2048
NBLK = NP // BLK

_f32 = jnp.float32


def _make_sc_spmv():
    mesh = plsc.VectorSubcoreMesh(core_axis_name="c", subcore_axis_name="s",
                                  num_cores=2, num_subcores=16)
    HC = CHUNKS // 2
    scratch = [
        pltpu.VMEM_SHARED((NP, HALF), _f32),
        pltpu.VMEM((HC, 128), jnp.int32),
        pltpu.VMEM((HC, 128), jnp.int32),
        pltpu.VMEM((128, HALF), _f32),
        pltpu.VMEM((128, HALF), _f32),
        pltpu.SemaphoreType.DMA,
        pltpu.SemaphoreType.DMA,
        pltpu.SemaphoreType.DMA,
    ]

    def body(table, init, srcs, dsts, out, acc, src_v, dst_v, g0, g1,
             s0, s1, ss):
        c = lax.axis_index("c")
        t = lax.axis_index("s")
        r0 = t * STRIPE

        NSTR = STRIPE // 128
        bufs = (g0, g1)
        pltpu.async_copy(init.at[c, pl.ds(r0, 128)], g0, s0)
        for h in range(NSTR):
            b = bufs[h % 2]
            sem = (s0, s1)[h % 2]
            if h + 1 < NSTR:
                pltpu.async_copy(init.at[c, pl.ds(r0 + (h + 1) * 128, 128)],
                                 bufs[(h + 1) % 2], (s0, s1)[(h + 1) % 2])
            pltpu.make_async_copy(init.at[c, pl.ds(r0 + h * 128, 128)],
                                  b, sem).wait()
            pltpu.sync_copy(b, acc.at[pl.ds(r0 + h * 128, 128)])
        plsc.subcore_barrier()

        tbl = table.at[c]

        for half in range(2):
            pltpu.sync_copy(srcs.at[t, pl.ds(half * HC, HC)], src_v)
            pltpu.sync_copy(dsts.at[t, pl.ds(half * HC, HC)], dst_v)
            pltpu.async_copy(tbl.at[src_v.at[0]], g0, s0)
            pltpu.async_copy(tbl.at[src_v.at[1]], g1, s1)

            def pipe(q, carry):
                j0 = 2 * q
                j1 = 2 * q + 1
                pltpu.make_async_copy(tbl.at[src_v.at[j0]], g0, s0).wait()
                d0 = pltpu.async_copy(g0, acc.at[dst_v.at[j0]], ss, add=True)
                pltpu.make_async_copy(tbl.at[src_v.at[j1]], g1, s1).wait()
                d1 = pltpu.async_copy(g1, acc.at[dst_v.at[j1]], ss, add=True)
                d0.wait()

                @pl.when(q < HC // 2 - 1)
                def _():
                    pltpu.async_copy(tbl.at[src_v.at[j0 + 2]], g0, s0)

                d1.wait()

                @pl.when(q < HC // 2 - 1)
                def _():
                    pltpu.async_copy(tbl.at[src_v.at[j1 + 2]], g1, s1)

                return carry

            lax.fori_loop(0, HC // 2, pipe, 0, unroll=False)

        plsc.subcore_barrier()

        pend = []
        for h in range(NSTR):
            rh = r0 + h * 128
            b = bufs[h % 2]
            if len(pend) >= 2:
                pend.pop(0).wait()
            pltpu.sync_copy(acc.at[pl.ds(rh, 128)], b)
            pend.append(pltpu.async_copy(b, out.at[c, pl.ds(rh, 128)], ss))
        for d in pend:
            d.wait()

    return pl.kernel(body,
                     out_type=(jax.ShapeDtypeStruct((2, NP, HALF), _f32),),
                     mesh=mesh, scratch_types=scratch)


def _make_sc_ea():
    mesh = plsc.VectorSubcoreMesh(core_axis_name="c", subcore_axis_name="s",
                                  num_cores=2, num_subcores=16)
    scratch = [
        pltpu.VMEM_SHARED((NP, HALF), _f32),
        pltpu.VMEM((EA_CHUNKS, 128), jnp.int32),
        pltpu.VMEM((128, HALF), _f32),
        pltpu.VMEM((128, HALF), _f32),
        pltpu.SemaphoreType.DMA,
        pltpu.SemaphoreType.DMA,
        pltpu.SemaphoreType.DMA,
    ]

    def body(ea_val, ea_dst, zeros128, out, acc, dst_v, vbuf, vbuf1,
             s0, s1, ss):
        c = lax.axis_index("c")
        t = lax.axis_index("s")
        r0 = t * STRIPE
        pltpu.sync_copy(ea_dst.at[c, t], dst_v)
        pltpu.sync_copy(zeros128, vbuf)

        def init_chunk(h, carry):
            pltpu.sync_copy(vbuf, acc.at[pl.ds(r0 + h * 128, 128)])
            return carry

        lax.fori_loop(0, STRIPE // 128, init_chunk, 0, unroll=False)
        plsc.subcore_barrier()
        NSTR = STRIPE // 128
        bufs = (vbuf, vbuf1)

        pltpu.async_copy(ea_val.at[c, t, pl.ds(0, 128)], vbuf, s0)
        pltpu.async_copy(ea_val.at[c, t, pl.ds(128, 128)], vbuf1, s1)

        def ea_pipe(q, carry):
            j0 = 2 * q
            j1 = 2 * q + 1
            pltpu.make_async_copy(
                ea_val.at[c, t, pl.ds(j0 * 128, 128)], vbuf, s0).wait()
            d0 = pltpu.async_copy(vbuf, acc.at[dst_v.at[j0]], ss, add=True)
            pltpu.make_async_copy(
                ea_val.at[c, t, pl.ds(j1 * 128, 128)], vbuf1, s1).wait()
            d1 = pltpu.async_copy(vbuf1, acc.at[dst_v.at[j1]], ss, add=True)
            d0.wait()

            @pl.when(q < EA_CHUNKS // 2 - 1)
            def _():
                pltpu.async_copy(
                    ea_val.at[c, t, pl.ds((j0 + 2) * 128, 128)], vbuf, s0)

            d1.wait()

            @pl.when(q < EA_CHUNKS // 2 - 1)
            def _():
                pltpu.async_copy(
                    ea_val.at[c, t, pl.ds((j1 + 2) * 128, 128)], vbuf1, s1)

            return carry

        lax.fori_loop(0, EA_CHUNKS // 2, ea_pipe, 0, unroll=False)
        plsc.subcore_barrier()

        pend = []
        for h in range(NSTR):
            rh = r0 + h * 128
            b = bufs[h % 2]
            if len(pend) >= 2:
                pend.pop(0).wait()
            pltpu.sync_copy(acc.at[pl.ds(rh, 128)], b)
            pend.append(pltpu.async_copy(b, out.at[c, pl.ds(rh, 128)], ss))
        for d in pend:
            d.wait()

    return pl.kernel(body,
                     out_type=(jax.ShapeDtypeStruct((2, NP, HALF), _f32),),
                     mesh=mesh, scratch_types=scratch)


@functools.lru_cache(maxsize=None)
def _sc_spmv():
    return _make_sc_spmv()


@functools.lru_cache(maxsize=None)
def _sc_ea():
    return _make_sc_ea()


def _tc_pre_body(x_ref, ea_ref, w1t_ref, w2t_ref, b_ref, y_ref, b0_ref, ch_ref):
    ea = ea_ref[0] + ea_ref[1]
    ct = jnp.dot(ea, w2t_ref[...], preferred_element_type=_f32,
                 precision=lax.Precision.HIGHEST) + b_ref[...]
    y0 = jnp.dot(x_ref[...], w1t_ref[...], preferred_element_type=_f32,
                 precision=lax.Precision.HIGHEST)
    b0 = y0 + ct
    y_ref[0] = y0[:, :HALF]
    y_ref[1] = y0[:, HALF:]
    b0_ref[0] = b0[:, :HALF]
    b0_ref[1] = b0[:, HALF:]
    ch_ref[0] = ct[:, :HALF]
    ch_ref[1] = ct[:, HALF:]


_tc_pre = pl.pallas_call(
    _tc_pre_body,
    grid=(NBLK,),
    in_specs=[
        pl.BlockSpec((BLK, D), lambda i: (i, 0)),
        pl.BlockSpec((2, BLK, DE), lambda i: (0, i, 0)),
        pl.BlockSpec((D, D), lambda i: (0, 0)),
        pl.BlockSpec((DE, D), lambda i: (0, 0)),
        pl.BlockSpec((1, D), lambda i: (0, 0)),
    ],
    out_specs=[
        pl.BlockSpec((2, BLK, HALF), lambda i: (0, i, 0)),
        pl.BlockSpec((2, BLK, HALF), lambda i: (0, i, 0)),
        pl.BlockSpec((2, BLK, HALF), lambda i: (0, i, 0)),
    ],
    out_shape=[
        jax.ShapeDtypeStruct((2, NP, HALF), _f32),
        jax.ShapeDtypeStruct((2, NP, HALF), _f32),
        jax.ShapeDtypeStruct((2, NP, HALF), _f32),
    ],
)


def _tc_conv_body(x1_ref, ch_ref, w1t_ref, y_ref, b1_ref):
    x1 = jnp.concatenate([x1_ref[0], x1_ref[1]], axis=-1)
    y1 = jnp.dot(x1, w1t_ref[...], preferred_element_type=_f32,
                 precision=lax.Precision.HIGHEST)
    y_ref[0] = y1[:, :HALF]
    y_ref[1] = y1[:, HALF:]
    b1_ref[0] = y1[:, :HALF] + ch_ref[0]
    b1_ref[1] = y1[:, HALF:] + ch_ref[1]


_tc_conv = pl.pallas_call(
    _tc_conv_body,
    grid=(NBLK,),
    in_specs=[
        pl.BlockSpec((2, BLK, HALF), lambda i: (0, i, 0)),
        pl.BlockSpec((2, BLK, HALF), lambda i: (0, i, 0)),
        pl.BlockSpec((D, D), lambda i: (0, 0)),
    ],
    out_specs=[
        pl.BlockSpec((2, BLK, HALF), lambda i: (0, i, 0)),
        pl.BlockSpec((2, BLK, HALF), lambda i: (0, i, 0)),
    ],
    out_shape=[
        jax.ShapeDtypeStruct((2, NP, HALF), _f32),
        jax.ShapeDtypeStruct((2, NP, HALF), _f32),
    ],
)


def _tc_pool_body(x_ref, batch_ref, wot_ref, b_ref,
                  wh1_ref, bh1_ref, wh2_ref, bh2_ref,
                  wp1_ref, bp1_ref, wp2_ref, bp2_ref,
                  mf_ref, o_ref, z_ref):
    x2 = jnp.concatenate([x_ref[0], x_ref[1]], axis=-1)
    l = jnp.dot(x2, wot_ref[...], preferred_element_type=_f32,
                precision=lax.Precision.HIGHEST) + b_ref[...]
    m = jnp.max(l, axis=1, keepdims=True)
    p = jnp.exp(l - m)
    a = p / jnp.sum(p, axis=1, keepdims=True)
    bt = batch_ref[...]
    gid = lax.broadcasted_iota(jnp.int32, (1, G), 1)
    oh = (bt == gid).astype(_f32)
    part = lax.dot_general(oh, a, (((0,), (0,)), ((), ())),
                           preferred_element_type=_f32,
                           precision=lax.Precision.HIGHEST)
    i = pl.program_id(0)

    @pl.when(i == 0)
    def _():
        mf_ref[...] = part

    @pl.when(i > 0)
    def _():
        mf_ref[...] += part

    @pl.when(i == NBLK - 1)
    def _():
        mf = mf_ref[...]
        h1 = jnp.maximum(
            jnp.dot(mf, wh1_ref[...], preferred_element_type=_f32,
                    precision=lax.Precision.HIGHEST) + bh1_ref[...], 0.0)
        z_ref[...] = jnp.dot(h1, wh2_ref[...], preferred_element_type=_f32,
                             precision=lax.Precision.HIGHEST) + bh2_ref[...]
        hid = jnp.dot(mf, wp1_ref[...], preferred_element_type=_f32,
                      precision=lax.Precision.HIGHEST) + bp1_ref[...]
        o = jnp.dot(hid, wp2_ref[...], preferred_element_type=_f32,
                    precision=lax.Precision.HIGHEST) + bp2_ref[...]
        o_ref[...] = jax.nn.sigmoid(o)


_tc_pool = pl.pallas_call(
    _tc_pool_body,
    grid=(NBLK,),
    in_specs=[
        pl.BlockSpec((2, BLK, HALF), lambda i: (0, i, 0)),
        pl.BlockSpec((BLK, 1), lambda i: (i, 0)),
        pl.BlockSpec((D, 512), lambda i: (0, 0)),
        pl.BlockSpec((1, 512), lambda i: (0, 0)),
        pl.BlockSpec((512, 128), lambda i: (0, 0)),
        pl.BlockSpec((1, 128), lambda i: (0, 0)),
        pl.BlockSpec((128, 1), lambda i: (0, 0)),
        pl.BlockSpec((1, 1), lambda i: (0, 0)),
        pl.BlockSpec((512, 50), lambda i: (0, 0)),
        pl.BlockSpec((1, 50), lambda i: (0, 0)),
        pl.BlockSpec((50, 1), lambda i: (0, 0)),
        pl.BlockSpec((1, 1), lambda i: (0, 0)),
    ],
    out_specs=[
        pl.BlockSpec((G, 512), lambda i: (0, 0)),
        pl.BlockSpec((G, 1), lambda i: (0, 0)),
        pl.BlockSpec((G, 1), lambda i: (0, 0)),
    ],
    out_shape=[
        jax.ShapeDtypeStruct((G, 512), _f32),
        jax.ShapeDtypeStruct((G, 1), _f32),
        jax.ShapeDtypeStruct((G, 1), _f32),
    ],
)


def kernel(x, edge_index, edge_attr, smiles, batch, is_supervised,
           W_in, b_in, W_out, b_out, W_p1, b_p1, W_p2, b_p2,
           W_h1, b_h1, W_h2, b_h2):
    del smiles, is_supervised
    xp = jnp.zeros((NP, D), _f32).at[:N].set(x)
    pad = EP - E
    srcp = jnp.concatenate(
        [edge_index[0], jnp.zeros((pad,), jnp.int32)]).reshape(16, CHUNKS, 128)
    dstp = jnp.concatenate(
        [edge_index[1], jnp.full((pad,), DUMMY, jnp.int32)]
    ).reshape(16, CHUNKS, 128)
    ea_dst = jnp.concatenate(
        [edge_index[1], jnp.zeros((pad,), jnp.int32)]
    ).reshape(2, 16, EA_CHUNKS, 128)
    ea128 = jnp.zeros((EP, HALF), _f32).at[:E, :DE].set(edge_attr)
    ea128 = ea128.reshape(2, 16, EP // 32, HALF)
    zeros128 = jnp.zeros((128, HALF), _f32)
    batchp = jnp.concatenate(
        [batch, jnp.full((NP - N,), G, jnp.int32)]).reshape(NP, 1)
    w1t = W_in[:, :D].T
    w2t = W_in[:, D:].T

    eap = _sc_ea()(ea128, ea_dst, zeros128)[0][:, :, :DE]
    y0, b0, ch = _tc_pre(xp, eap, w1t, w2t, b_in.reshape(1, D))
    x1 = _sc_spmv()(y0, b0, srcp, dstp)[0]
    y1, b1 = _tc_conv(x1, ch, w1t)
    x2 = _sc_spmv()(y1, b1, srcp, dstp)[0]
    _, o, z = _tc_pool(x2, batchp, W_out.T, b_out.reshape(1, 512),
                       W_h1.T, b_h1.reshape(1, -1), W_h2.T,
                       b_h2.reshape(1, -1), W_p1.T, b_p1.reshape(1, -1),
                       W_p2.T, b_p2.reshape(1, -1))
    return (o, z)
